# baseline (speedup 1.0000x reference)
import math

import jax
import jax.numpy as jnp
from jax import lax
from jax.experimental import pallas as pl
from jax.experimental.pallas import tpu as pltpu
from jax.experimental.pallas import tpu_sc as plsc

D = 64
SCALE = math.sqrt(D)

_NC = 2
_NS = 16
_NW = _NC * _NS
_L = 16


def _iota16():
    return lax.iota(jnp.int32, _L)


def _make_lookup(B: int, H: int):
    nblk = B // 128
    assert nblk % _NW == 0 and H % 2 == 0, (B, H)
    bpw = nblk // _NW
    mesh = plsc.VectorSubcoreMesh(core_axis_name="c", subcore_axis_name="s")

    def body(pk_hbm, xT_hbm, out_hbm, idx_v, pidx0, pidx1, qc0, qc1,
             rows0, rows1, tr0, tr1, gsem0, gsem1, ssem0, ssem1):
        wid = lax.axis_index("s") * _NC + lax.axis_index("c")
        bufs = ((pidx0, qc0, rows0, tr0, gsem0, ssem0),
                (pidx1, qc1, rows1, tr1, gsem1, ssem1))

        def prep(h, buf):
            pidx, qc = buf[0], buf[1]
            for k in range(8):
                sl = pl.ds(16 * k, _L)
                iv = idx_v[h, sl]
                pidx[sl] = lax.shift_right_logical(iv, 1)
                qc[sl] = (iv & 1) * 64

        def fire_gather(buf):
            pltpu.async_copy(pk_hbm.at[buf[0]], buf[2], buf[4])

        def wait_gather(buf):
            pltpu.make_async_copy(pk_hbm.at[pl.ds(0, 128)], buf[2],
                                  buf[4]).wait()

        def transpose(buf):
            qc, rows, tr = buf[1], buf[2], buf[3]
            for k in range(8):
                rvec = jnp.int32(16 * k) + _iota16()
                qck = qc[pl.ds(16 * k, _L)]

                @pl.loop(0, D, unroll=8)
                def _d(d):
                    vals = plsc.load_gather(rows, [rvec, qck + d])
                    tr[d, pl.ds(16 * k, _L)] = vals * SCALE

        def fire_store(h, b0, buf):
            pltpu.async_copy(buf[3], out_hbm.at[h, :, pl.ds(b0, 128)],
                             buf[5])

        def wait_store(buf):
            pltpu.make_async_copy(buf[3], out_hbm.at[0, :, pl.ds(0, 128)],
                                  buf[5]).wait()

        @pl.loop(0, bpw)
        def _blk(blk):
            b0 = (wid * bpw + blk) * 128
            pltpu.sync_copy(xT_hbm.at[:, pl.ds(b0, 128)], idx_v)

            prep(0, bufs[0])
            fire_gather(bufs[0])
            prep(1, bufs[1])
            fire_gather(bufs[1])

            @pl.loop(0, H, step=2)
            def _h(h):
                for par in range(2):
                    hh = h + par
                    buf = bufs[par]
                    wait_gather(buf)

                    @pl.when(hh >= 2)
                    def _():
                        wait_store(buf)

                    transpose(buf)
                    fire_store(hh, b0, buf)

                    @pl.when(hh + 2 < H)
                    def _():
                        prep(hh + 2, buf)
                        fire_gather(buf)

            wait_store(bufs[0])
            wait_store(bufs[1])

    return pl.kernel(
        body,
        out_type=jax.ShapeDtypeStruct((H, D, B), jnp.float32),
        mesh=mesh,
        scratch_types=[
            pltpu.VMEM((H, 128), jnp.int32),
            pltpu.VMEM((128,), jnp.int32),
            pltpu.VMEM((128,), jnp.int32),
            pltpu.VMEM((128,), jnp.int32),
            pltpu.VMEM((128,), jnp.int32),
            pltpu.VMEM((128, 128), jnp.float32),
            pltpu.VMEM((128, 128), jnp.float32),
            pltpu.VMEM((D, 128), jnp.float32),
            pltpu.VMEM((D, 128), jnp.float32),
            pltpu.SemaphoreType.DMA,
            pltpu.SemaphoreType.DMA,
            pltpu.SemaphoreType.DMA,
            pltpu.SemaphoreType.DMA,
        ],
        compiler_params=pltpu.CompilerParams(needs_layout_passes=False),
    )


def kernel(x, table):
    batch, hist = x.shape
    xT = x.T.astype(jnp.int32)
    packed = table.reshape(-1, 128)
    out3 = _make_lookup(batch, hist)(packed, xT)
    return out3.transpose(2, 0, 1)
